# Initial kernel scaffold; baseline (speedup 1.0000x reference)
import jax
import jax.numpy as jnp
from jax.experimental import pallas as pl


def kernel(params, node_s, node_v, edge_s, edge_v, edge_index):
    raise NotImplementedError("write your pallas kernel here")


# speedup vs baseline: 8.8424x; 8.8424x over previous
import jax
import jax.numpy as jnp
from jax import lax
from jax.experimental import pallas as pl
from jax.experimental.pallas import tpu as pltpu
from jax.experimental.pallas import tpu_sc as plsc

N = 10000
E = 160000
TW = 384
TILE_E = 640
TILE_N = 1000
GCH = 40
SCH = 80

f32 = jnp.float32


def _kron3(w):
    a, b = w.shape
    return (jnp.eye(3, dtype=f32)[:, None, :, None] * w[None, :, None, :]).reshape(3 * a, 3 * b)


def _clipss(x, h):
    s = x * x
    return jnp.clip(s[:, 0:h] + s[:, h:2 * h] + s[:, 2 * h:3 * h], 1e-8, None)


def _gate(vt, h):
    sig = jax.nn.sigmoid(jnp.sqrt(_clipss(vt, h)))
    return vt * jnp.concatenate([sig, sig, sig], axis=1)


def _ln_math(w, b, s, vt):
    mu = jnp.mean(s, axis=-1, keepdims=True)
    var = jnp.mean(jnp.square(s - mu), axis=-1, keepdims=True)
    s = (s - mu) / jnp.sqrt(var + 1e-5) * w + b
    ss = _clipss(vt, 3)
    vn = jnp.sqrt(jnp.mean(ss, axis=-1, keepdims=True))
    return s, vt / vn


def _prep_conv(plist, si):
    g0, g1, g2 = plist
    ws, wh = g0['ws'], g0['wh']
    K = 16 if si == 6 else 272
    Wsrc = jnp.zeros((K, TW), f32)
    Wsrc = Wsrc.at[0:si, 0:256].set(ws[0:si])
    Wsrc = Wsrc.at[si:si + 9, 256:277].set(_kron3(wh[0:3]))
    Wdst = jnp.zeros((K, TW), f32)
    Wdst = Wdst.at[0:si, 0:256].set(ws[si + 32:2 * si + 32])
    Wdst = Wdst.at[si:si + 9, 256:277].set(_kron3(wh[4:7]))
    We = jnp.zeros((35, TW), f32)
    We = We.at[0:32, 0:256].set(ws[si:si + 32])
    We = We.at[32:35, 256:277].set(_kron3(wh[3:4]))
    d = {
        'Wsrc': Wsrc, 'Wdst': Wdst, 'We': We,
        'wvn0': ws[2 * si + 32:2 * si + 39],
        'bs0': g0['bs'][None, :],
        'wv0k': _kron3(g0['wv']),
    }
    for i, g in ((1, g1), (2, g2)):
        d[f'ws{i}'] = g['ws'][:256]
        d[f'wvn{i}'] = g['ws'][256:259]
        d[f'bs{i}'] = g['bs'][None, :]
        d[f'wh{i}k'] = _kron3(g['wh'])
        d[f'wv{i}k'] = _kron3(g['wv'])
    return d


def _table_body(x_ref, ws_ref, wd_ref, ts_ref, td_ref):
    x = x_ref[...]
    ts_ref[...] = jnp.dot(x, ws_ref[...], preferred_element_type=f32)
    td_ref[...] = jnp.dot(x, wd_ref[...], preferred_element_type=f32)


def _table_call(X, Wsrc, Wdst):
    K = X.shape[1]
    return pl.pallas_call(
        _table_body,
        grid=(N // TILE_N,),
        in_specs=[pl.BlockSpec((TILE_N, K), lambda i: (i, 0)),
                  pl.BlockSpec((K, TW), lambda i: (0, 0)),
                  pl.BlockSpec((K, TW), lambda i: (0, 0))],
        out_specs=[pl.BlockSpec((TILE_N, TW), lambda i: (i, 0)),
                   pl.BlockSpec((TILE_N, TW), lambda i: (i, 0))],
        out_shape=[jax.ShapeDtypeStruct((N, TW), f32)] * 2,
    )(X, Wsrc, Wdst)


def _edge_body(gs_ref, gd_ref, ev_ref, we_ref, wvn0_ref, b0_ref, wv0_ref,
               w1_ref, wvn1_ref, b1_ref, wh1_ref, wv1_ref,
               w2_ref, wvn2_ref, b2_ref, wh2_ref, wv2_ref, f1_ref, f2_ref):
    M = gs_ref[...] + gd_ref[...] + jnp.dot(ev_ref[...], we_ref[...],
                                            preferred_element_type=f32)
    vh0 = M[:, 256:277]
    vn0 = jnp.sqrt(_clipss(vh0, 7))
    s0 = jax.nn.relu(M[:, 0:256] + jnp.dot(vn0, wvn0_ref[...]) + b0_ref[...])
    vt1 = _gate(jnp.dot(vh0, wv0_ref[...]), 3)
    vh1 = jnp.dot(vt1, wh1_ref[...])
    vn1 = jnp.sqrt(_clipss(vh1, 3))
    s1 = jax.nn.relu(jnp.dot(s0, w1_ref[...], preferred_element_type=f32)
                     + jnp.dot(vn1, wvn1_ref[...]) + b1_ref[...])
    vt2 = _gate(jnp.dot(vh1, wv1_ref[...]), 3)
    vh2 = jnp.dot(vt2, wh2_ref[...])
    vn2 = jnp.sqrt(_clipss(vh2, 3))
    s2 = (jnp.dot(s1, w2_ref[...], preferred_element_type=f32)
          + jnp.dot(vn2, wvn2_ref[...]) + b2_ref[...])
    vt3 = jnp.dot(vh2, wv2_ref[...])
    f1_ref[...] = s2
    f2_ref[...] = jnp.concatenate(
        [vt3, jnp.ones((TILE_E, 1), f32),
         jnp.zeros((TILE_E, 118), f32)], axis=1)


def _edge_call(Gs, Gd, EV, d):
    full = lambda shape: pl.BlockSpec(shape, lambda i: tuple(0 for _ in shape))
    return pl.pallas_call(
        _edge_body,
        grid=(E // TILE_E,),
        in_specs=[pl.BlockSpec((TILE_E, TW), lambda i: (i, 0)),
                  pl.BlockSpec((TILE_E, TW), lambda i: (i, 0)),
                  pl.BlockSpec((TILE_E, 35), lambda i: (i, 0)),
                  full((35, TW)), full((7, 256)), full((1, 256)), full((21, 9)),
                  full((256, 256)), full((3, 256)), full((1, 256)), full((9, 9)), full((9, 9)),
                  full((256, 256)), full((3, 256)), full((1, 256)), full((9, 9)), full((9, 9))],
        out_specs=[pl.BlockSpec((TILE_E, 256), lambda i: (i, 0)),
                   pl.BlockSpec((TILE_E, 128), lambda i: (i, 0))],
        out_shape=[jax.ShapeDtypeStruct((E, 256), f32),
                   jax.ShapeDtypeStruct((E, 128), f32)],
    )(Gs, Gd, EV, d['We'], d['wvn0'], d['bs0'], d['wv0k'],
      d['ws1'], d['wvn1'], d['bs1'], d['wh1k'], d['wv1k'],
      d['ws2'], d['wvn2'], d['bs2'], d['wh2k'], d['wv2k'])


def _unpack_acc(a0_ref, a1_ref, e_ref):
    s_sum = jnp.concatenate([a0_ref[0], a1_ref[0]], axis=1)
    e = e_ref[...]
    deg = jnp.clip(e[:, 9:10], 1.0, None)
    return s_sum / deg, e[:, 0:9] / deg


def _acc_specs():
    return [pl.BlockSpec((1, TILE_N, 128), lambda i: (0, i, 0)),
            pl.BlockSpec((1, TILE_N, 128), lambda i: (1, i, 0)),
            pl.BlockSpec((TILE_N, 128), lambda i: (i, 0))]


def _node1_body(a0_ref, a1_ref, e_ref, w_ref, b_ref, x_ref):
    s, vt = _unpack_acc(a0_ref, a1_ref, e_ref)
    s, vt = _ln_math(w_ref[...], b_ref[...], s, vt)
    x_ref[...] = jnp.concatenate([s, vt, jnp.zeros((TILE_N, 7), f32)], axis=1)


def _node1_call(accS, accE, ln):
    return pl.pallas_call(
        _node1_body,
        grid=(N // TILE_N,),
        in_specs=_acc_specs() + [
                  pl.BlockSpec((1, 256), lambda i: (0, 0)),
                  pl.BlockSpec((1, 256), lambda i: (0, 0))],
        out_specs=[pl.BlockSpec((TILE_N, 272), lambda i: (i, 0))],
        out_shape=[jax.ShapeDtypeStruct((N, 272), f32)],
    )(accS, accS, accE, ln['weight'][None, :], ln['bias'][None, :])[0]


def _node2_body(a0_ref, a1_ref, e_ref, xp_ref, n0w_ref, n0b_ref,
                wf1_ref, wvnf1_ref, bf1_ref, whf1_ref, wvf1_ref,
                wf2_ref, wvnf2_ref, bf2_ref, whf2_ref, wvf2_ref,
                n1w_ref, n1b_ref, lnw_ref, lnb_ref, x_ref):
    dhs, dhvt = _unpack_acc(a0_ref, a1_ref, e_ref)
    s_prev = xp_ref[...][:, 0:256]
    vt_prev = xp_ref[...][:, 256:265]
    s, vt = _ln_math(n0w_ref[...], n0b_ref[...], s_prev + dhs, vt_prev + dhvt)
    vh = jnp.dot(vt, whf1_ref[...])
    vnf = jnp.sqrt(_clipss(vh, 6))
    sf = jax.nn.relu(jnp.dot(s, wf1_ref[...], preferred_element_type=f32)
                     + jnp.dot(vnf, wvnf1_ref[...]) + bf1_ref[...])
    vtf = _gate(jnp.dot(vh, wvf1_ref[...]), 6)
    vh2 = jnp.dot(vtf, whf2_ref[...])
    vn2 = jnp.sqrt(_clipss(vh2, 6))
    s2 = (jnp.dot(sf, wf2_ref[...], preferred_element_type=f32)
          + jnp.dot(vn2, wvnf2_ref[...]) + bf2_ref[...])
    vt2 = jnp.dot(vh2, wvf2_ref[...])
    s, vt = _ln_math(n1w_ref[...], n1b_ref[...], s + s2, vt + vt2)
    s, vt = _ln_math(lnw_ref[...], lnb_ref[...], s, vt)
    x_ref[...] = jnp.concatenate([s, vt, jnp.zeros((TILE_N, 7), f32)], axis=1)


def _node2_call(accS, accE, Xprev, lay, ln):
    g1, g2 = lay['ff']
    full = lambda shape: pl.BlockSpec(shape, lambda i: tuple(0 for _ in shape))
    row = lambda: full((1, 256))
    return pl.pallas_call(
        _node2_body,
        grid=(N // TILE_N,),
        in_specs=_acc_specs() + [
                  pl.BlockSpec((TILE_N, 272), lambda i: (i, 0)),
                  row(), row(),
                  full((256, 1024)), full((6, 1024)), full((1, 1024)),
                  full((9, 18)), full((18, 18)),
                  full((1024, 256)), full((6, 256)), full((1, 256)),
                  full((18, 18)), full((18, 9)),
                  row(), row(), row(), row()],
        out_specs=[pl.BlockSpec((TILE_N, 272), lambda i: (i, 0))],
        out_shape=[jax.ShapeDtypeStruct((N, 272), f32)],
    )(accS, accS, accE, Xprev,
      lay['norm0']['weight'][None, :], lay['norm0']['bias'][None, :],
      g1['ws'][:256], g1['ws'][256:262], g1['bs'][None, :],
      _kron3(g1['wh']), _kron3(g1['wv']),
      g2['ws'][:1024], g2['ws'][1024:1030], g2['bs'][None, :],
      _kron3(g2['wh']), _kron3(g2['wv']),
      lay['norm1']['weight'][None, :], lay['norm1']['bias'][None, :],
      ln['weight'][None, :], ln['bias'][None, :])[0]


def _head_body(x_ref, mw_ref, mb_ref, o_ref, acc_ref):
    i = pl.program_id(0)

    @pl.when(i == 0)
    def _():
        acc_ref[...] = jnp.zeros_like(acc_ref)

    acc_ref[...] += jnp.sum(x_ref[...][:, 0:256], axis=0, keepdims=True)

    @pl.when(i == pl.num_programs(0) - 1)
    def _():
        o_ref[...] = (jnp.dot(acc_ref[...] / N, mw_ref[...],
                              preferred_element_type=f32) + mb_ref[...])


def _head_call(X, mw, mb):
    return pl.pallas_call(
        _head_body,
        grid=(N // TILE_N,),
        in_specs=[pl.BlockSpec((TILE_N, 272), lambda i: (i, 0)),
                  pl.BlockSpec((256, 250), lambda i: (0, 0)),
                  pl.BlockSpec((1, 250), lambda i: (0, 0))],
        out_specs=[pl.BlockSpec((1, 250), lambda i: (0, 0))],
        out_shape=[jax.ShapeDtypeStruct((1, 250), f32)],
        scratch_shapes=[pltpu.VMEM((1, 256), f32)],
    )(X, mw, mb[None, :])[0]


def _sc_mesh():
    return plsc.VectorSubcoreMesh(core_axis_name="c", subcore_axis_name="s",
                                  num_cores=2, num_subcores=16)


def _gather_body(ts_hbm, td_hbm, src_hbm, dst_hbm, gs_hbm, gd_hbm,
                 sidx, didx, bufs, bufd, sems, semd):
    wid = lax.axis_index("s") * 2 + lax.axis_index("c")
    pltpu.sync_copy(src_hbm.at[wid], sidx)
    pltpu.sync_copy(dst_hbm.at[wid], didx)
    base = wid * (E // 32)

    def step(j, carry):
        pltpu.async_copy(ts_hbm.at[sidx.at[j]], bufs, sems).wait()
        pltpu.sync_copy(bufs, gs_hbm.at[pl.ds(base + j * GCH, GCH)])
        pltpu.async_copy(td_hbm.at[didx.at[j]], bufd, semd).wait()
        pltpu.sync_copy(bufd, gd_hbm.at[pl.ds(base + j * GCH, GCH)])
        return carry

    lax.fori_loop(0, E // 32 // GCH, step, 0)


def _sc_gather(Ts, Td, src3, dst3):
    nch = E // 32 // GCH
    kfn = pl.kernel(
        _gather_body,
        out_type=[jax.ShapeDtypeStruct((E, TW), f32)] * 2,
        mesh=_sc_mesh(),
        scratch_types=[pltpu.VMEM((nch, GCH), jnp.int32),
                       pltpu.VMEM((nch, GCH), jnp.int32),
                       pltpu.VMEM((GCH, TW), f32),
                       pltpu.VMEM((GCH, TW), f32),
                       pltpu.SemaphoreType.DMA,
                       pltpu.SemaphoreType.DMA])
    return kfn(Ts, Td, src3, dst3)


def _scatter_body(f1_hbm, f2_hbm, dst_hbm, z_hbm, accs_hbm, acce_hbm,
                  idx, buf, shared):
    cid = lax.axis_index("c")
    sid = lax.axis_index("s")
    nch = E // 16 // SCH

    def copy_rows(src_at, dst_at):
        @pl.when(sid < 15)
        def _():
            r = pl.ds(sid * 624, 624)
            pltpu.sync_copy(src_at(r), dst_at(r))

        @pl.when(sid == 15)
        def _():
            r = pl.ds(9360, 640)
            pltpu.sync_copy(src_at(r), dst_at(r))

    pltpu.sync_copy(dst_hbm.at[sid], idx)
    copy_rows(lambda r: z_hbm.at[r], lambda r: shared.at[r])
    plsc.subcore_barrier()
    base = sid * (E // 16)

    def step0(j, carry):
        pltpu.sync_copy(
            f1_hbm.at[pl.ds(base + j * SCH, SCH), pl.ds(cid * 128, 128)], buf)
        pltpu.sync_copy(buf, shared.at[idx.at[j]], add=True)
        return carry

    lax.fori_loop(0, nch, step0, 0)
    plsc.subcore_barrier()
    copy_rows(lambda r: shared.at[r], lambda r: accs_hbm.at[cid, r])
    plsc.subcore_barrier()

    @pl.when(cid == 0)
    def _():
        copy_rows(lambda r: z_hbm.at[r], lambda r: shared.at[r])

    plsc.subcore_barrier()

    @pl.when(cid == 0)
    def _():
        def step1(j, carry):
            pltpu.sync_copy(f2_hbm.at[pl.ds(base + j * SCH, SCH)], buf)
            pltpu.sync_copy(buf, shared.at[idx.at[j]], add=True)
            return carry

        lax.fori_loop(0, nch, step1, 0)

    plsc.subcore_barrier()

    @pl.when(cid == 0)
    def _():
        copy_rows(lambda r: shared.at[r], lambda r: acce_hbm.at[r])


def _sc_scatter(F1, F2, dst_sc, z128):
    nch = E // 16 // SCH
    kfn = pl.kernel(
        _scatter_body,
        out_type=[jax.ShapeDtypeStruct((2, N, 128), f32),
                  jax.ShapeDtypeStruct((N, 128), f32)],
        mesh=_sc_mesh(),
        scratch_types=[pltpu.VMEM((nch, SCH), jnp.int32),
                       pltpu.VMEM((SCH, 128), f32),
                       pltpu.VMEM_SHARED((N, 128), f32)])
    return kfn(F1, F2, dst_sc, z128)


def kernel(params, node_s, node_v, edge_s, edge_v, edge_index):
    src = edge_index[0]
    dst = edge_index[1]
    src3 = src.reshape(32, E // 32 // GCH, GCH)
    dst3 = dst.reshape(32, E // 32 // GCH, GCH)
    dst_sc = dst.reshape(16, E // 16 // SCH, SCH)
    EV = jnp.concatenate([edge_s, edge_v[:, 0, :]], axis=1)
    vt_node = jnp.swapaxes(node_v, -1, -2).reshape(N, 9)
    X = jnp.concatenate([node_s, vt_node, jnp.zeros((N, 1), f32)], axis=1)
    z128 = jnp.zeros((N, 128), f32)

    d0 = _prep_conv(params['conv0'], 6)
    Ts, Td = _table_call(X, d0['Wsrc'], d0['Wdst'])
    Gs, Gd = _sc_gather(Ts, Td, src3, dst3)
    F1, F2 = _edge_call(Gs, Gd, EV, d0)
    accS, accE = _sc_scatter(F1, F2, dst_sc, z128)
    X = _node1_call(accS, accE, params['ln'])

    for name in ('conv1', 'conv2'):
        lay = params[name]
        dl = _prep_conv(lay['conv'], 256)
        Ts, Td = _table_call(X, dl['Wsrc'], dl['Wdst'])
        Gs, Gd = _sc_gather(Ts, Td, src3, dst3)
        F1, F2 = _edge_call(Gs, Gd, EV, dl)
        accS, accE = _sc_scatter(F1, F2, dst_sc, z128)
        X = _node2_call(accS, accE, X, lay, params['ln'])

    out = _head_call(X, params['mean_w'], params['mean_b'])
    return out.reshape(250)


# speedup vs baseline: 9.5799x; 1.0834x over previous
import jax
import jax.numpy as jnp
from jax import lax
from jax.experimental import pallas as pl
from jax.experimental.pallas import tpu as pltpu
from jax.experimental.pallas import tpu_sc as plsc

N = 10000
E = 160000
TW = 384
TILE_E = 640
TILE_N = 1000
GCH = 72
GNC = 70
EPW = 5000
SCH = 80

f32 = jnp.float32
bf16 = jnp.bfloat16


def _kron3(w):
    a, b = w.shape
    return (jnp.eye(3, dtype=f32)[:, None, :, None] * w[None, :, None, :]).reshape(3 * a, 3 * b)


def _clipss(x, h):
    s = x * x
    return jnp.clip(s[:, 0:h] + s[:, h:2 * h] + s[:, 2 * h:3 * h], 1e-8, None)


def _gate(vt, h):
    sig = jax.nn.sigmoid(jnp.sqrt(_clipss(vt, h)))
    return vt * jnp.concatenate([sig, sig, sig], axis=1)


def _ln_math(w, b, s, vt):
    mu = jnp.mean(s, axis=-1, keepdims=True)
    var = jnp.mean(jnp.square(s - mu), axis=-1, keepdims=True)
    s = (s - mu) / jnp.sqrt(var + 1e-5) * w + b
    ss = _clipss(vt, 3)
    vn = jnp.sqrt(jnp.mean(ss, axis=-1, keepdims=True))
    return s, vt / vn


def _prep_conv(plist, si):
    g0, g1, g2 = plist
    ws, wh = g0['ws'], g0['wh']
    K = 16 if si == 6 else 272
    Wsrc = jnp.zeros((K, TW), f32)
    Wsrc = Wsrc.at[0:si, 0:256].set(ws[0:si])
    Wsrc = Wsrc.at[si:si + 9, 256:277].set(_kron3(wh[0:3]))
    Wdst = jnp.zeros((K, TW), f32)
    Wdst = Wdst.at[0:si, 0:256].set(ws[si + 32:2 * si + 32])
    Wdst = Wdst.at[si:si + 9, 256:277].set(_kron3(wh[4:7]))
    We = jnp.zeros((35, TW), f32)
    We = We.at[0:32, 0:256].set(ws[si:si + 32])
    We = We.at[32:35, 256:277].set(_kron3(wh[3:4]))
    d = {
        'Wsrc': Wsrc, 'Wdst': Wdst, 'We': We,
        'wvn0': ws[2 * si + 32:2 * si + 39],
        'bs0': g0['bs'][None, :],
        'wv0k': _kron3(g0['wv']),
    }
    for i, g in ((1, g1), (2, g2)):
        d[f'ws{i}'] = g['ws'][:256]
        d[f'wvn{i}'] = g['ws'][256:259]
        d[f'bs{i}'] = g['bs'][None, :]
        d[f'wh{i}k'] = _kron3(g['wh'])
        d[f'wv{i}k'] = _kron3(g['wv'])
    return d


def _table_body(x_ref, ws_ref, wd_ref, ts_ref, td_ref):
    x = x_ref[...]
    ts_ref[...] = jnp.dot(x, ws_ref[...], preferred_element_type=f32)
    td_ref[...] = jnp.dot(x, wd_ref[...], preferred_element_type=f32)


def _table_call(X, Wsrc, Wdst):
    K = X.shape[1]
    return pl.pallas_call(
        _table_body,
        grid=(N // TILE_N,),
        in_specs=[pl.BlockSpec((TILE_N, K), lambda i: (i, 0)),
                  pl.BlockSpec((K, TW), lambda i: (0, 0)),
                  pl.BlockSpec((K, TW), lambda i: (0, 0))],
        out_specs=[pl.BlockSpec((TILE_N, TW), lambda i: (i, 0)),
                   pl.BlockSpec((TILE_N, TW), lambda i: (i, 0))],
        out_shape=[jax.ShapeDtypeStruct((N, TW), f32)] * 2,
    )(X, Wsrc, Wdst)


def _edge_body(gs_ref, gd_ref, ev_ref, we_ref, wvn0_ref, b0_ref, wv0_ref,
               w1_ref, wvn1_ref, b1_ref, wh1_ref, wv1_ref,
               w2_ref, wvn2_ref, b2_ref, wh2_ref, wv2_ref, f1_ref, f2_ref):
    M = (gs_ref[...] + gd_ref[...]
         + jnp.dot(ev_ref[...], we_ref[...], preferred_element_type=f32))
    vh0 = M[:, 256:277]
    vn0 = jnp.sqrt(_clipss(vh0, 7))
    s0 = jax.nn.relu(M[:, 0:256] + jnp.dot(vn0, wvn0_ref[...]) + b0_ref[...])
    vt1 = _gate(jnp.dot(vh0, wv0_ref[...]), 3)
    vh1 = jnp.dot(vt1, wh1_ref[...])
    vn1 = jnp.sqrt(_clipss(vh1, 3))
    s1 = jax.nn.relu(jnp.dot(s0, w1_ref[...], preferred_element_type=f32)
                     + jnp.dot(vn1, wvn1_ref[...]) + b1_ref[...])
    vt2 = _gate(jnp.dot(vh1, wv1_ref[...]), 3)
    vh2 = jnp.dot(vt2, wh2_ref[...])
    vn2 = jnp.sqrt(_clipss(vh2, 3))
    s2 = (jnp.dot(s1, w2_ref[...], preferred_element_type=f32)
          + jnp.dot(vn2, wvn2_ref[...]) + b2_ref[...])
    vt3 = jnp.dot(vh2, wv2_ref[...])
    f1_ref[...] = s2
    f2_ref[...] = jnp.concatenate(
        [vt3, jnp.ones((TILE_E, 1), f32),
         jnp.zeros((TILE_E, 118), f32)], axis=1)


def _edge_call(Gs, Gd, EV, d):
    full = lambda shape: pl.BlockSpec(shape, lambda i: tuple(0 for _ in shape))
    return pl.pallas_call(
        _edge_body,
        grid=(E // TILE_E,),
        in_specs=[pl.BlockSpec((TILE_E, TW), lambda i: (i, 0)),
                  pl.BlockSpec((TILE_E, TW), lambda i: (i, 0)),
                  pl.BlockSpec((TILE_E, 35), lambda i: (i, 0)),
                  full((35, TW)), full((7, 256)), full((1, 256)), full((21, 9)),
                  full((256, 256)), full((3, 256)), full((1, 256)), full((9, 9)), full((9, 9)),
                  full((256, 256)), full((3, 256)), full((1, 256)), full((9, 9)), full((9, 9))],
        out_specs=[pl.BlockSpec((TILE_E, 256), lambda i: (i, 0)),
                   pl.BlockSpec((TILE_E, 128), lambda i: (i, 0))],
        out_shape=[jax.ShapeDtypeStruct((E, 256), f32),
                   jax.ShapeDtypeStruct((E, 128), f32)],
    )(Gs, Gd, EV, d['We'], d['wvn0'], d['bs0'], d['wv0k'],
      d['ws1'], d['wvn1'], d['bs1'], d['wh1k'], d['wv1k'],
      d['ws2'], d['wvn2'], d['bs2'], d['wh2k'], d['wv2k'])


def _unpack_acc(a0_ref, a1_ref, e_ref):
    s_sum = jnp.concatenate([a0_ref[0], a1_ref[0]], axis=1)
    e = e_ref[...]
    deg = jnp.clip(e[:, 9:10], 1.0, None)
    return s_sum / deg, e[:, 0:9] / deg


def _acc_specs():
    return [pl.BlockSpec((1, TILE_N, 128), lambda i: (0, i, 0)),
            pl.BlockSpec((1, TILE_N, 128), lambda i: (1, i, 0)),
            pl.BlockSpec((TILE_N, 128), lambda i: (i, 0))]


def _node1_body(a0_ref, a1_ref, e_ref, w_ref, b_ref, x_ref):
    s, vt = _unpack_acc(a0_ref, a1_ref, e_ref)
    s, vt = _ln_math(w_ref[...], b_ref[...], s, vt)
    x_ref[...] = jnp.concatenate([s, vt, jnp.zeros((TILE_N, 7), f32)], axis=1)


def _node1_call(accS, accE, ln):
    return pl.pallas_call(
        _node1_body,
        grid=(N // TILE_N,),
        in_specs=_acc_specs() + [
                  pl.BlockSpec((1, 256), lambda i: (0, 0)),
                  pl.BlockSpec((1, 256), lambda i: (0, 0))],
        out_specs=[pl.BlockSpec((TILE_N, 272), lambda i: (i, 0))],
        out_shape=[jax.ShapeDtypeStruct((N, 272), f32)],
    )(accS, accS, accE, ln['weight'][None, :], ln['bias'][None, :])[0]


def _node2_body(a0_ref, a1_ref, e_ref, xp_ref, n0w_ref, n0b_ref,
                wf1_ref, wvnf1_ref, bf1_ref, whf1_ref, wvf1_ref,
                wf2_ref, wvnf2_ref, bf2_ref, whf2_ref, wvf2_ref,
                n1w_ref, n1b_ref, lnw_ref, lnb_ref, x_ref):
    dhs, dhvt = _unpack_acc(a0_ref, a1_ref, e_ref)
    s_prev = xp_ref[...][:, 0:256]
    vt_prev = xp_ref[...][:, 256:265]
    s, vt = _ln_math(n0w_ref[...], n0b_ref[...], s_prev + dhs, vt_prev + dhvt)
    vh = jnp.dot(vt, whf1_ref[...])
    vnf = jnp.sqrt(_clipss(vh, 6))
    sf = jax.nn.relu(jnp.dot(s, wf1_ref[...], preferred_element_type=f32)
                     + jnp.dot(vnf, wvnf1_ref[...]) + bf1_ref[...])
    vtf = _gate(jnp.dot(vh, wvf1_ref[...]), 6)
    vh2 = jnp.dot(vtf, whf2_ref[...])
    vn2 = jnp.sqrt(_clipss(vh2, 6))
    s2 = (jnp.dot(sf, wf2_ref[...], preferred_element_type=f32)
          + jnp.dot(vn2, wvnf2_ref[...]) + bf2_ref[...])
    vt2 = jnp.dot(vh2, wvf2_ref[...])
    s, vt = _ln_math(n1w_ref[...], n1b_ref[...], s + s2, vt + vt2)
    s, vt = _ln_math(lnw_ref[...], lnb_ref[...], s, vt)
    x_ref[...] = jnp.concatenate([s, vt, jnp.zeros((TILE_N, 7), f32)], axis=1)


def _node2_call(accS, accE, Xprev, lay, ln):
    g1, g2 = lay['ff']
    full = lambda shape: pl.BlockSpec(shape, lambda i: tuple(0 for _ in shape))
    row = lambda: full((1, 256))
    return pl.pallas_call(
        _node2_body,
        grid=(N // TILE_N,),
        in_specs=_acc_specs() + [
                  pl.BlockSpec((TILE_N, 272), lambda i: (i, 0)),
                  row(), row(),
                  full((256, 1024)), full((6, 1024)), full((1, 1024)),
                  full((9, 18)), full((18, 18)),
                  full((1024, 256)), full((6, 256)), full((1, 256)),
                  full((18, 18)), full((18, 9)),
                  row(), row(), row(), row()],
        out_specs=[pl.BlockSpec((TILE_N, 272), lambda i: (i, 0))],
        out_shape=[jax.ShapeDtypeStruct((N, 272), f32)],
    )(accS, accS, accE, Xprev,
      lay['norm0']['weight'][None, :], lay['norm0']['bias'][None, :],
      g1['ws'][:256], g1['ws'][256:262], g1['bs'][None, :],
      _kron3(g1['wh']), _kron3(g1['wv']),
      g2['ws'][:1024], g2['ws'][1024:1030], g2['bs'][None, :],
      _kron3(g2['wh']), _kron3(g2['wv']),
      lay['norm1']['weight'][None, :], lay['norm1']['bias'][None, :],
      ln['weight'][None, :], ln['bias'][None, :])[0]


def _head_body(x_ref, mw_ref, mb_ref, o_ref, acc_ref):
    i = pl.program_id(0)

    @pl.when(i == 0)
    def _():
        acc_ref[...] = jnp.zeros_like(acc_ref)

    acc_ref[...] += jnp.sum(x_ref[...][:, 0:256], axis=0, keepdims=True)

    @pl.when(i == pl.num_programs(0) - 1)
    def _():
        o_ref[...] = (jnp.dot(acc_ref[...] / N, mw_ref[...],
                              preferred_element_type=f32) + mb_ref[...])


def _head_call(X, mw, mb):
    return pl.pallas_call(
        _head_body,
        grid=(N // TILE_N,),
        in_specs=[pl.BlockSpec((TILE_N, 272), lambda i: (i, 0)),
                  pl.BlockSpec((256, 250), lambda i: (0, 0)),
                  pl.BlockSpec((1, 250), lambda i: (0, 0))],
        out_specs=[pl.BlockSpec((1, 250), lambda i: (0, 0))],
        out_shape=[jax.ShapeDtypeStruct((1, 250), f32)],
        scratch_shapes=[pltpu.VMEM((1, 256), f32)],
    )(X, mw, mb[None, :])[0]


def _sc_mesh():
    return plsc.VectorSubcoreMesh(core_axis_name="c", subcore_axis_name="s",
                                  num_cores=2, num_subcores=16)


def _gather_body(ts_hbm, td_hbm, src_hbm, dst_hbm, gs_hbm, gd_hbm,
                 sidx, didx, bs0, bs1, bd0, bd1, ss0, ss1, sd0, sd1):
    wid = lax.axis_index("s") * 2 + lax.axis_index("c")
    pltpu.sync_copy(src_hbm.at[wid], sidx)
    pltpu.sync_copy(dst_hbm.at[wid], didx)
    base = wid * EPW
    bufs = ((bs0, bd0, ss0, sd0), (bs1, bd1, ss1, sd1))

    def issue(j, b):
        bs, bd, ss, sd = bufs[b]
        pltpu.async_copy(ts_hbm.at[sidx.at[j]], bs, ss)
        pltpu.async_copy(td_hbm.at[didx.at[j]], bd, sd)

    def drain_and_write(j, b):
        bs, bd, ss, sd = bufs[b]
        pltpu.make_async_copy(ts_hbm.at[sidx.at[j]], bs, ss).wait()
        pltpu.make_async_copy(td_hbm.at[didx.at[j]], bd, sd).wait()

        @pl.when(j < GNC - 1)
        def _():
            o = pl.ds(base + j * GCH, GCH)
            pltpu.sync_copy(bs, gs_hbm.at[o])
            pltpu.sync_copy(bd, gd_hbm.at[o])

        @pl.when(j == GNC - 1)
        def _():
            o = pl.ds(base + j * GCH, EPW - (GNC - 1) * GCH)
            pltpu.sync_copy(bs.at[pl.ds(0, EPW - (GNC - 1) * GCH)], gs_hbm.at[o])
            pltpu.sync_copy(bd.at[pl.ds(0, EPW - (GNC - 1) * GCH)], gd_hbm.at[o])

    issue(0, 0)

    def step(j2, carry):
        for b in range(2):
            j = j2 * 2 + b

            @pl.when(j + 1 < GNC)
            def _():
                issue(j + 1, 1 - b)

            drain_and_write(j, b)
        return carry

    lax.fori_loop(0, GNC // 2, step, 0)


def _sc_gather(Ts, Td, src3, dst3):
    kfn = pl.kernel(
        _gather_body,
        out_type=[jax.ShapeDtypeStruct((E, TW), f32)] * 2,
        mesh=_sc_mesh(),
        scratch_types=[pltpu.VMEM((GNC, GCH), jnp.int32),
                       pltpu.VMEM((GNC, GCH), jnp.int32),
                       pltpu.VMEM((GCH, TW), f32),
                       pltpu.VMEM((GCH, TW), f32),
                       pltpu.VMEM((GCH, TW), f32),
                       pltpu.VMEM((GCH, TW), f32),
                       pltpu.SemaphoreType.DMA,
                       pltpu.SemaphoreType.DMA,
                       pltpu.SemaphoreType.DMA,
                       pltpu.SemaphoreType.DMA])
    return kfn(Ts, Td, src3, dst3)


def _scatter_body(f1_hbm, f2_hbm, dst_hbm, z_hbm, accs_hbm, acce_hbm,
                  idx, buf, shared):
    cid = lax.axis_index("c")
    sid = lax.axis_index("s")
    nch = E // 16 // SCH

    def copy_rows(src_at, dst_at):
        @pl.when(sid < 15)
        def _():
            r = pl.ds(sid * 624, 624)
            pltpu.sync_copy(src_at(r), dst_at(r))

        @pl.when(sid == 15)
        def _():
            r = pl.ds(9360, 640)
            pltpu.sync_copy(src_at(r), dst_at(r))

    pltpu.sync_copy(dst_hbm.at[sid], idx)
    copy_rows(lambda r: z_hbm.at[r], lambda r: shared.at[r])
    plsc.subcore_barrier()
    base = sid * (E // 16)

    def step0(j, carry):
        pltpu.sync_copy(
            f1_hbm.at[pl.ds(base + j * SCH, SCH), pl.ds(cid * 128, 128)], buf)
        pltpu.sync_copy(buf, shared.at[idx.at[j]], add=True)
        return carry

    lax.fori_loop(0, nch, step0, 0)
    plsc.subcore_barrier()
    copy_rows(lambda r: shared.at[r], lambda r: accs_hbm.at[cid, r])
    plsc.subcore_barrier()

    @pl.when(cid == 0)
    def _():
        copy_rows(lambda r: z_hbm.at[r], lambda r: shared.at[r])

    plsc.subcore_barrier()

    @pl.when(cid == 0)
    def _():
        def step1(j, carry):
            pltpu.sync_copy(f2_hbm.at[pl.ds(base + j * SCH, SCH)], buf)
            pltpu.sync_copy(buf, shared.at[idx.at[j]], add=True)
            return carry

        lax.fori_loop(0, nch, step1, 0)

    plsc.subcore_barrier()

    @pl.when(cid == 0)
    def _():
        copy_rows(lambda r: shared.at[r], lambda r: acce_hbm.at[r])


def _sc_scatter(F1, F2, dst_sc, z128):
    nch = E // 16 // SCH
    kfn = pl.kernel(
        _scatter_body,
        out_type=[jax.ShapeDtypeStruct((2, N, 128), f32),
                  jax.ShapeDtypeStruct((N, 128), f32)],
        mesh=_sc_mesh(),
        scratch_types=[pltpu.VMEM((nch, SCH), jnp.int32),
                       pltpu.VMEM((SCH, 128), f32),
                       pltpu.VMEM_SHARED((N, 128), f32)])
    return kfn(F1, F2, dst_sc, z128)


def kernel(params, node_s, node_v, edge_s, edge_v, edge_index):
    src = edge_index[0]
    dst = edge_index[1]
    pad = GNC * GCH - EPW
    src3 = jnp.pad(src.reshape(32, EPW), ((0, 0), (0, pad))).reshape(32, GNC, GCH)
    dst3 = jnp.pad(dst.reshape(32, EPW), ((0, 0), (0, pad))).reshape(32, GNC, GCH)
    dst_sc = dst.reshape(16, E // 16 // SCH, SCH)
    EV = jnp.concatenate([edge_s, edge_v[:, 0, :]], axis=1)
    vt_node = jnp.swapaxes(node_v, -1, -2).reshape(N, 9)
    X = jnp.concatenate([node_s, vt_node, jnp.zeros((N, 1), f32)], axis=1)
    z128 = jnp.zeros((N, 128), f32)

    d0 = _prep_conv(params['conv0'], 6)
    Ts, Td = _table_call(X, d0['Wsrc'], d0['Wdst'])
    Gs, Gd = _sc_gather(Ts, Td, src3, dst3)
    F1, F2 = _edge_call(Gs, Gd, EV, d0)
    accS, accE = _sc_scatter(F1, F2, dst_sc, z128)
    X = _node1_call(accS, accE, params['ln'])

    for name in ('conv1', 'conv2'):
        lay = params[name]
        dl = _prep_conv(lay['conv'], 256)
        Ts, Td = _table_call(X, dl['Wsrc'], dl['Wdst'])
        Gs, Gd = _sc_gather(Ts, Td, src3, dst3)
        F1, F2 = _edge_call(Gs, Gd, EV, dl)
        accS, accE = _sc_scatter(F1, F2, dst_sc, z128)
        X = _node2_call(accS, accE, X, lay, params['ln'])

    out = _head_call(X, params['mean_w'], params['mean_b'])
    return out.reshape(250)


# speedup vs baseline: 10.9156x; 1.1394x over previous
import jax
import jax.numpy as jnp
from jax import lax
from jax.experimental import pallas as pl
from jax.experimental.pallas import tpu as pltpu
from jax.experimental.pallas import tpu_sc as plsc

N = 10000
E = 160000
TW = 384
TILE_E = 640
TILE_N = 1000
GCH = 72
GNC = 70
EPW = 5000
SCH = 80

f32 = jnp.float32
bf16 = jnp.bfloat16


def _kron3(w):
    a, b = w.shape
    return (jnp.eye(3, dtype=f32)[:, None, :, None] * w[None, :, None, :]).reshape(3 * a, 3 * b)


def _clipss(x, h):
    s = x * x
    return jnp.clip(s[:, 0:h] + s[:, h:2 * h] + s[:, 2 * h:3 * h], 1e-8, None)


def _gate(vt, h):
    sig = jax.nn.sigmoid(jnp.sqrt(_clipss(vt, h)))
    return vt * jnp.concatenate([sig, sig, sig], axis=1)


def _ln_math(w, b, s, vt):
    mu = jnp.mean(s, axis=-1, keepdims=True)
    var = jnp.mean(jnp.square(s - mu), axis=-1, keepdims=True)
    s = (s - mu) / jnp.sqrt(var + 1e-5) * w + b
    ss = _clipss(vt, 3)
    vn = jnp.sqrt(jnp.mean(ss, axis=-1, keepdims=True))
    return s, vt / vn


def _prep_conv(plist, si):
    g0, g1, g2 = plist
    ws, wh = g0['ws'], g0['wh']
    K = 16 if si == 6 else 272
    Wsrc = jnp.zeros((K, TW), f32)
    Wsrc = Wsrc.at[0:si, 0:256].set(ws[0:si])
    Wsrc = Wsrc.at[si:si + 9, 256:277].set(_kron3(wh[0:3]))
    Wdst = jnp.zeros((K, TW), f32)
    Wdst = Wdst.at[0:si, 0:256].set(ws[si + 32:2 * si + 32])
    Wdst = Wdst.at[si:si + 9, 256:277].set(_kron3(wh[4:7]))
    We = jnp.zeros((35, TW), f32)
    We = We.at[0:32, 0:256].set(ws[si:si + 32])
    We = We.at[32:35, 256:277].set(_kron3(wh[3:4]))
    d = {
        'Wsrc': Wsrc, 'Wdst': Wdst, 'We': We,
        'wvn0': ws[2 * si + 32:2 * si + 39],
        'bs0': g0['bs'][None, :],
        'wv0k': _kron3(g0['wv']),
    }
    for i, g in ((1, g1), (2, g2)):
        d[f'ws{i}'] = g['ws'][:256]
        d[f'wvn{i}'] = g['ws'][256:259]
        d[f'bs{i}'] = g['bs'][None, :]
        d[f'wh{i}k'] = _kron3(g['wh'])
        d[f'wv{i}k'] = _kron3(g['wv'])
    return d


def _table_body(x_ref, ws_ref, wd_ref, ts_ref, td_ref):
    x = x_ref[...]
    ts_ref[...] = jnp.dot(x, ws_ref[...], preferred_element_type=f32)
    td_ref[...] = jnp.dot(x, wd_ref[...], preferred_element_type=f32)


def _table_call(X, Wsrc, Wdst):
    K = X.shape[1]
    return pl.pallas_call(
        _table_body,
        grid=(N // TILE_N,),
        in_specs=[pl.BlockSpec((TILE_N, K), lambda i: (i, 0)),
                  pl.BlockSpec((K, TW), lambda i: (0, 0)),
                  pl.BlockSpec((K, TW), lambda i: (0, 0))],
        out_specs=[pl.BlockSpec((TILE_N, TW), lambda i: (i, 0)),
                   pl.BlockSpec((TILE_N, TW), lambda i: (i, 0))],
        out_shape=[jax.ShapeDtypeStruct((N, TW), f32)] * 2,
    )(X, Wsrc, Wdst)


def _edge_body(gs_ref, gd_ref, ev_ref, we_ref, wvn0_ref, b0_ref, wv0_ref,
               w1_ref, wvn1_ref, b1_ref, wh1_ref, wv1_ref,
               w2_ref, wvn2_ref, b2_ref, wh2_ref, wv2_ref, f1_ref, f2_ref):
    M = (gs_ref[...] + gd_ref[...]
         + jnp.dot(ev_ref[...], we_ref[...], preferred_element_type=f32))
    vh0 = M[:, 256:277]
    vn0 = jnp.sqrt(_clipss(vh0, 7))
    s0 = jax.nn.relu(M[:, 0:256] + jnp.dot(vn0, wvn0_ref[...]) + b0_ref[...])
    vt1 = _gate(jnp.dot(vh0, wv0_ref[...]), 3)
    vh1 = jnp.dot(vt1, wh1_ref[...])
    vn1 = jnp.sqrt(_clipss(vh1, 3))
    s1 = jax.nn.relu(jnp.dot(s0, w1_ref[...], preferred_element_type=f32)
                     + jnp.dot(vn1, wvn1_ref[...]) + b1_ref[...])
    vt2 = _gate(jnp.dot(vh1, wv1_ref[...]), 3)
    vh2 = jnp.dot(vt2, wh2_ref[...])
    vn2 = jnp.sqrt(_clipss(vh2, 3))
    s2 = (jnp.dot(s1, w2_ref[...], preferred_element_type=f32)
          + jnp.dot(vn2, wvn2_ref[...]) + b2_ref[...])
    vt3 = jnp.dot(vh2, wv2_ref[...])
    f1_ref[...] = s2
    f2_ref[...] = jnp.concatenate(
        [vt3, jnp.ones((TILE_E, 1), f32),
         jnp.zeros((TILE_E, 118), f32)], axis=1)


def _edge_call(Gs, Gd, EV, d):
    full = lambda shape: pl.BlockSpec(shape, lambda i: tuple(0 for _ in shape))
    return pl.pallas_call(
        _edge_body,
        grid=(E // TILE_E,),
        in_specs=[pl.BlockSpec((TILE_E, TW), lambda i: (i, 0)),
                  pl.BlockSpec((TILE_E, TW), lambda i: (i, 0)),
                  pl.BlockSpec((TILE_E, 35), lambda i: (i, 0)),
                  full((35, TW)), full((7, 256)), full((1, 256)), full((21, 9)),
                  full((256, 256)), full((3, 256)), full((1, 256)), full((9, 9)), full((9, 9)),
                  full((256, 256)), full((3, 256)), full((1, 256)), full((9, 9)), full((9, 9))],
        out_specs=[pl.BlockSpec((TILE_E, 256), lambda i: (i, 0)),
                   pl.BlockSpec((TILE_E, 128), lambda i: (i, 0))],
        out_shape=[jax.ShapeDtypeStruct((E, 256), f32),
                   jax.ShapeDtypeStruct((E, 128), f32)],
    )(Gs, Gd, EV, d['We'], d['wvn0'], d['bs0'], d['wv0k'],
      d['ws1'], d['wvn1'], d['bs1'], d['wh1k'], d['wv1k'],
      d['ws2'], d['wvn2'], d['bs2'], d['wh2k'], d['wv2k'])


def _unpack_acc(a0_ref, a1_ref, e0_ref, e1_ref):
    s_sum = jnp.concatenate([a0_ref[0], a1_ref[0]], axis=1)
    e = e0_ref[0] + e1_ref[0]
    deg = jnp.clip(e[:, 9:10], 1.0, None)
    return s_sum / deg, e[:, 0:9] / deg


def _acc_specs():
    return [pl.BlockSpec((1, TILE_N, 128), lambda i: (0, i, 0)),
            pl.BlockSpec((1, TILE_N, 128), lambda i: (1, i, 0)),
            pl.BlockSpec((1, TILE_N, 128), lambda i: (0, i, 0)),
            pl.BlockSpec((1, TILE_N, 128), lambda i: (1, i, 0))]


def _node1_body(a0_ref, a1_ref, e0_ref, e1_ref, w_ref, b_ref, x_ref):
    s, vt = _unpack_acc(a0_ref, a1_ref, e0_ref, e1_ref)
    s, vt = _ln_math(w_ref[...], b_ref[...], s, vt)
    x_ref[...] = jnp.concatenate([s, vt, jnp.zeros((TILE_N, 7), f32)], axis=1)


def _node1_call(accS, accE, ln):
    return pl.pallas_call(
        _node1_body,
        grid=(N // TILE_N,),
        in_specs=_acc_specs() + [
                  pl.BlockSpec((1, 256), lambda i: (0, 0)),
                  pl.BlockSpec((1, 256), lambda i: (0, 0))],
        out_specs=[pl.BlockSpec((TILE_N, 272), lambda i: (i, 0))],
        out_shape=[jax.ShapeDtypeStruct((N, 272), f32)],
    )(accS, accS, accE, accE, ln['weight'][None, :], ln['bias'][None, :])[0]


def _node2_body(a0_ref, a1_ref, e0_ref, e1_ref, xp_ref, n0w_ref, n0b_ref,
                wf1_ref, wvnf1_ref, bf1_ref, whf1_ref, wvf1_ref,
                wf2_ref, wvnf2_ref, bf2_ref, whf2_ref, wvf2_ref,
                n1w_ref, n1b_ref, lnw_ref, lnb_ref, x_ref):
    dhs, dhvt = _unpack_acc(a0_ref, a1_ref, e0_ref, e1_ref)
    s_prev = xp_ref[...][:, 0:256]
    vt_prev = xp_ref[...][:, 256:265]
    s, vt = _ln_math(n0w_ref[...], n0b_ref[...], s_prev + dhs, vt_prev + dhvt)
    vh = jnp.dot(vt, whf1_ref[...])
    vnf = jnp.sqrt(_clipss(vh, 6))
    sf = jax.nn.relu(jnp.dot(s, wf1_ref[...], preferred_element_type=f32)
                     + jnp.dot(vnf, wvnf1_ref[...]) + bf1_ref[...])
    vtf = _gate(jnp.dot(vh, wvf1_ref[...]), 6)
    vh2 = jnp.dot(vtf, whf2_ref[...])
    vn2 = jnp.sqrt(_clipss(vh2, 6))
    s2 = (jnp.dot(sf, wf2_ref[...], preferred_element_type=f32)
          + jnp.dot(vn2, wvnf2_ref[...]) + bf2_ref[...])
    vt2 = jnp.dot(vh2, wvf2_ref[...])
    s, vt = _ln_math(n1w_ref[...], n1b_ref[...], s + s2, vt + vt2)
    s, vt = _ln_math(lnw_ref[...], lnb_ref[...], s, vt)
    x_ref[...] = jnp.concatenate([s, vt, jnp.zeros((TILE_N, 7), f32)], axis=1)


def _node2_call(accS, accE, Xprev, lay, ln):
    g1, g2 = lay['ff']
    full = lambda shape: pl.BlockSpec(shape, lambda i: tuple(0 for _ in shape))
    row = lambda: full((1, 256))
    return pl.pallas_call(
        _node2_body,
        grid=(N // TILE_N,),
        in_specs=_acc_specs() + [
                  pl.BlockSpec((TILE_N, 272), lambda i: (i, 0)),
                  row(), row(),
                  full((256, 1024)), full((6, 1024)), full((1, 1024)),
                  full((9, 18)), full((18, 18)),
                  full((1024, 256)), full((6, 256)), full((1, 256)),
                  full((18, 18)), full((18, 9)),
                  row(), row(), row(), row()],
        out_specs=[pl.BlockSpec((TILE_N, 272), lambda i: (i, 0))],
        out_shape=[jax.ShapeDtypeStruct((N, 272), f32)],
    )(accS, accS, accE, accE, Xprev,
      lay['norm0']['weight'][None, :], lay['norm0']['bias'][None, :],
      g1['ws'][:256], g1['ws'][256:262], g1['bs'][None, :],
      _kron3(g1['wh']), _kron3(g1['wv']),
      g2['ws'][:1024], g2['ws'][1024:1030], g2['bs'][None, :],
      _kron3(g2['wh']), _kron3(g2['wv']),
      lay['norm1']['weight'][None, :], lay['norm1']['bias'][None, :],
      ln['weight'][None, :], ln['bias'][None, :])[0]


def _head_body(x_ref, mw_ref, mb_ref, o_ref, acc_ref):
    i = pl.program_id(0)

    @pl.when(i == 0)
    def _():
        acc_ref[...] = jnp.zeros_like(acc_ref)

    acc_ref[...] += jnp.sum(x_ref[...][:, 0:256], axis=0, keepdims=True)

    @pl.when(i == pl.num_programs(0) - 1)
    def _():
        o_ref[...] = (jnp.dot(acc_ref[...] / N, mw_ref[...],
                              preferred_element_type=f32) + mb_ref[...])


def _head_call(X, mw, mb):
    return pl.pallas_call(
        _head_body,
        grid=(N // TILE_N,),
        in_specs=[pl.BlockSpec((TILE_N, 272), lambda i: (i, 0)),
                  pl.BlockSpec((256, 250), lambda i: (0, 0)),
                  pl.BlockSpec((1, 250), lambda i: (0, 0))],
        out_specs=[pl.BlockSpec((1, 250), lambda i: (0, 0))],
        out_shape=[jax.ShapeDtypeStruct((1, 250), f32)],
        scratch_shapes=[pltpu.VMEM((1, 256), f32)],
    )(X, mw, mb[None, :])[0]


def _sc_mesh():
    return plsc.VectorSubcoreMesh(core_axis_name="c", subcore_axis_name="s",
                                  num_cores=2, num_subcores=16)


def _gather_body(ts_hbm, td_hbm, src_hbm, dst_hbm, gs_hbm, gd_hbm,
                 sidx, didx, bs0, bs1, bd0, bd1, ss0, ss1, sd0, sd1):
    wid = lax.axis_index("s") * 2 + lax.axis_index("c")
    pltpu.sync_copy(src_hbm.at[wid], sidx)
    pltpu.sync_copy(dst_hbm.at[wid], didx)
    base = wid * EPW
    bufs = ((bs0, bd0, ss0, sd0), (bs1, bd1, ss1, sd1))

    def issue(j, b):
        bs, bd, ss, sd = bufs[b]
        pltpu.async_copy(ts_hbm.at[sidx.at[j]], bs, ss)
        pltpu.async_copy(td_hbm.at[didx.at[j]], bd, sd)

    def drain_and_write(j, b):
        bs, bd, ss, sd = bufs[b]
        pltpu.make_async_copy(ts_hbm.at[sidx.at[j]], bs, ss).wait()
        pltpu.make_async_copy(td_hbm.at[didx.at[j]], bd, sd).wait()

        @pl.when(j < GNC - 1)
        def _():
            o = pl.ds(base + j * GCH, GCH)
            pltpu.sync_copy(bs, gs_hbm.at[o])
            pltpu.sync_copy(bd, gd_hbm.at[o])

        @pl.when(j == GNC - 1)
        def _():
            o = pl.ds(base + j * GCH, EPW - (GNC - 1) * GCH)
            pltpu.sync_copy(bs.at[pl.ds(0, EPW - (GNC - 1) * GCH)], gs_hbm.at[o])
            pltpu.sync_copy(bd.at[pl.ds(0, EPW - (GNC - 1) * GCH)], gd_hbm.at[o])

    issue(0, 0)

    def step(j2, carry):
        for b in range(2):
            j = j2 * 2 + b

            @pl.when(j + 1 < GNC)
            def _():
                issue(j + 1, 1 - b)

            drain_and_write(j, b)
        return carry

    lax.fori_loop(0, GNC // 2, step, 0)


def _sc_gather(Ts, Td, src3, dst3):
    kfn = pl.kernel(
        _gather_body,
        out_type=[jax.ShapeDtypeStruct((E, TW), f32)] * 2,
        mesh=_sc_mesh(),
        scratch_types=[pltpu.VMEM((GNC, GCH), jnp.int32),
                       pltpu.VMEM((GNC, GCH), jnp.int32),
                       pltpu.VMEM((GCH, TW), f32),
                       pltpu.VMEM((GCH, TW), f32),
                       pltpu.VMEM((GCH, TW), f32),
                       pltpu.VMEM((GCH, TW), f32),
                       pltpu.SemaphoreType.DMA,
                       pltpu.SemaphoreType.DMA,
                       pltpu.SemaphoreType.DMA,
                       pltpu.SemaphoreType.DMA])
    return kfn(Ts, Td, src3, dst3)


def _scatter_body(f1_hbm, f2_hbm, dst_hbm, z_hbm, accs_hbm, acce_hbm,
                  idx, b0, b1, s0, s1, shared):
    cid = lax.axis_index("c")
    sid = lax.axis_index("s")
    nch = E // 16 // SCH

    def copy_rows(src_at, dst_at):
        @pl.when(sid < 15)
        def _():
            r = pl.ds(sid * 624, 624)
            pltpu.sync_copy(src_at(r), dst_at(r))

        @pl.when(sid == 15)
        def _():
            r = pl.ds(9360, 640)
            pltpu.sync_copy(src_at(r), dst_at(r))

    pltpu.sync_copy(dst_hbm.at[sid], idx)
    copy_rows(lambda r: z_hbm.at[r], lambda r: shared.at[r])
    plsc.subcore_barrier()
    base = sid * (E // 16)
    bufs = ((b0, s0), (b1, s1))

    def src0(j):
        return f1_hbm.at[pl.ds(base + j * SCH, SCH), pl.ds(cid * 128, 128)]

    def issue0(j, b):
        pltpu.async_copy(src0(j), bufs[b][0], bufs[b][1])

    def drain0(j, b):
        pltpu.make_async_copy(src0(j), bufs[b][0], bufs[b][1]).wait()
        pltpu.sync_copy(bufs[b][0], shared.at[idx.at[j]], add=True)

    issue0(0, 0)

    def step0(j2, carry):
        for b in range(2):
            j = j2 * 2 + b

            @pl.when(j + 1 < nch)
            def _():
                issue0(j + 1, 1 - b)

            drain0(j, b)
        return carry

    lax.fori_loop(0, nch // 2, step0, 0)
    drain0(nch - 1, (nch - 1) % 2)
    plsc.subcore_barrier()
    copy_rows(lambda r: shared.at[r], lambda r: accs_hbm.at[cid, r])
    plsc.subcore_barrier()
    copy_rows(lambda r: z_hbm.at[r], lambda r: shared.at[r])
    plsc.subcore_barrier()

    start = cid * 63
    end = 63 + cid * (nch - 63)

    def step1(j, carry):
        pltpu.sync_copy(f2_hbm.at[pl.ds(base + j * SCH, SCH)], b0)
        pltpu.sync_copy(b0, shared.at[idx.at[j]], add=True)
        return carry

    lax.fori_loop(start, end, step1, 0)
    plsc.subcore_barrier()
    copy_rows(lambda r: shared.at[r], lambda r: acce_hbm.at[cid, r])


def _sc_scatter(F1, F2, dst_sc, z128):
    nch = E // 16 // SCH
    kfn = pl.kernel(
        _scatter_body,
        out_type=[jax.ShapeDtypeStruct((2, N, 128), f32),
                  jax.ShapeDtypeStruct((2, N, 128), f32)],
        mesh=_sc_mesh(),
        scratch_types=[pltpu.VMEM((nch, SCH), jnp.int32),
                       pltpu.VMEM((SCH, 128), f32),
                       pltpu.VMEM((SCH, 128), f32),
                       pltpu.SemaphoreType.DMA,
                       pltpu.SemaphoreType.DMA,
                       pltpu.VMEM_SHARED((N, 128), f32)])
    return kfn(F1, F2, dst_sc, z128)


def kernel(params, node_s, node_v, edge_s, edge_v, edge_index):
    src = edge_index[0]
    dst = edge_index[1]
    pad = GNC * GCH - EPW
    src3 = jnp.pad(src.reshape(32, EPW), ((0, 0), (0, pad))).reshape(32, GNC, GCH)
    dst3 = jnp.pad(dst.reshape(32, EPW), ((0, 0), (0, pad))).reshape(32, GNC, GCH)
    dst_sc = dst.reshape(16, E // 16 // SCH, SCH)
    EV = jnp.concatenate([edge_s, edge_v[:, 0, :]], axis=1)
    vt_node = jnp.swapaxes(node_v, -1, -2).reshape(N, 9)
    X = jnp.concatenate([node_s, vt_node, jnp.zeros((N, 1), f32)], axis=1)
    z128 = jnp.zeros((N, 128), f32)

    d0 = _prep_conv(params['conv0'], 6)
    Ts, Td = _table_call(X, d0['Wsrc'], d0['Wdst'])
    Gs, Gd = _sc_gather(Ts, Td, src3, dst3)
    F1, F2 = _edge_call(Gs, Gd, EV, d0)
    accS, accE = _sc_scatter(F1, F2, dst_sc, z128)
    X = _node1_call(accS, accE, params['ln'])

    for name in ('conv1', 'conv2'):
        lay = params[name]
        dl = _prep_conv(lay['conv'], 256)
        Ts, Td = _table_call(X, dl['Wsrc'], dl['Wdst'])
        Gs, Gd = _sc_gather(Ts, Td, src3, dst3)
        F1, F2 = _edge_call(Gs, Gd, EV, dl)
        accS, accE = _sc_scatter(F1, F2, dst_sc, z128)
        X = _node2_call(accS, accE, X, lay, params['ln'])

    out = _head_call(X, params['mean_w'], params['mean_b'])
    return out.reshape(250)
